# Initial kernel scaffold; baseline (speedup 1.0000x reference)
import jax
import jax.numpy as jnp
from jax.experimental import pallas as pl


def kernel(E_user, E_gender, E_item, E_cate, Wa1, ba1, Wa2, ba2, Wa3, ba3, gamma, beta, Wf1, bf1, alpha1, Wf2, bf2, alpha2, Wf3, bf3, user_id, gender, target_item_id, target_cate_id, hist_item_id, hist_cate_id, length):
    raise NotImplementedError("write your pallas kernel here")


# speedup vs baseline: 6.5506x; 6.5506x over previous
import jax
import jax.numpy as jnp
from jax import lax
from jax.experimental import pallas as pl
from jax.experimental.pallas import tpu as pltpu
from jax.experimental.pallas import tpu_sc as plsc

B = 4096
T = 200
D = 64
BT = B * T
NC = 2
NS = 16
NW = NC * NS
R = BT // 128 // NW
BB_A = 64
BB_B = 512


def _sc_gather(eu, eg, ei, ec, uid2, gid2, tid2, cid2, hii2, hci2,
               up_out, tgt_out, hist_out, idx_b, rows_b, sem):
    wid = lax.axis_index("s") * NC + lax.axis_index("c")
    for ta, ia, tb, ib, out in ((eu, uid2, eg, gid2, up_out),
                                (ei, tid2, ec, cid2, tgt_out)):
        pltpu.sync_copy(ia.at[pl.ds(wid, 1)], idx_b.at[pl.ds(0, 1)])
        pltpu.sync_copy(ib.at[pl.ds(wid, 1)], idx_b.at[pl.ds(1, 1)])
        pltpu.async_copy(ta.at[idx_b.at[0]], rows_b, sem).wait()
        pltpu.async_copy(tb.at[idx_b.at[1]], rows_b, sem, add=True).wait()
        pltpu.sync_copy(rows_b, out.at[pl.ds(wid * 128, 128)])
    pltpu.sync_copy(hii2.at[pl.ds(wid * R, R)], idx_b.at[pl.ds(0, R)])
    pltpu.sync_copy(hci2.at[pl.ds(wid * R, R)], idx_b.at[pl.ds(R, R)])

    def body(j, carry):
        base = (wid * R + j) * 128
        pltpu.async_copy(ei.at[idx_b.at[j]], rows_b, sem).wait()
        pltpu.async_copy(ec.at[idx_b.at[R + j]], rows_b, sem, add=True).wait()
        pltpu.sync_copy(rows_b, hist_out.at[pl.ds(base, 128)])
        return carry

    lax.fori_loop(0, R, body, 0)


def _run_sc_gather(EU, EG, EI, EC, uid2, gid2, tid2, cid2, hii2, hci2):
    mesh = plsc.VectorSubcoreMesh(core_axis_name="c", subcore_axis_name="s")
    f32 = jnp.float32
    call = pl.kernel(
        _sc_gather, mesh=mesh,
        out_type=[
            jax.ShapeDtypeStruct((B, 128), f32),
            jax.ShapeDtypeStruct((B, 128), f32),
            jax.ShapeDtypeStruct((BT, 128), f32),
        ],
        scratch_types=[
            pltpu.VMEM((2 * R, 128), jnp.int32),
            pltpu.VMEM((128, 128), f32),
            pltpu.SemaphoreType.DMA,
        ],
    )
    return call(EU, EG, EI, EC, uid2, gid2, tid2, cid2, hii2, hci2)


def _attn_body(h_ref, up_ref, tgt_ref, len_ref,
               w1t_ref, wqd_ref, ba1_ref, w2t_ref, ba2_ref, wa3_ref,
               ba3_ref, join_ref, stats_ref):
    Bb = 64
    N = Bb * T
    h3 = h_ref[...].reshape(T, Bb, 128)
    q = tgt_ref[...]
    p3 = h3 * q[None, :, :]
    X = jnp.concatenate([h3, p3], axis=2).reshape(N, 256)
    XT = X.T
    bias = jnp.dot(q, wqd_ref[...], preferred_element_type=jnp.float32)
    biasT = jnp.broadcast_to(bias[None, :, :], (T, Bb, 80))
    biasT = biasT.reshape(N, 80).T
    a1 = jnp.dot(w1t_ref[...], XT, preferred_element_type=jnp.float32)
    a1 = jax.nn.sigmoid(a1 + biasT + ba1_ref[...])
    a2 = jnp.dot(w2t_ref[...], a1, preferred_element_type=jnp.float32)
    a2 = jax.nn.sigmoid(a2 + ba2_ref[...])
    s = jnp.sum(a2 * wa3_ref[...], axis=0)
    s100 = s.reshape(N // 128, 128) + ba3_ref[0, 0]
    row_i = lax.broadcasted_iota(jnp.int32, (N // 128, 128), 0)
    col_i = lax.broadcasted_iota(jnp.int32, (N // 128, 128), 1)
    t_mat = 2 * row_i + col_i // 64
    mask = t_mat < len_ref[...].reshape(1, 128)
    sm = jnp.where(mask, s100, jnp.float32(-(2.0 ** 32) + 1.0))
    m1 = jnp.max(sm, axis=0, keepdims=True)
    m2 = jnp.maximum(m1[:, :64], m1[:, 64:])
    e = jnp.exp(sm - jnp.concatenate([m2, m2], axis=1))
    s1 = jnp.sum(e, axis=0, keepdims=True)
    d2 = s1[:, :64] + s1[:, 64:]
    wgt = e / jnp.concatenate([d2, d2], axis=1)
    Xh3 = XT[0:128, :].reshape(128, N // 128, 128)
    acc = jnp.sum(Xh3 * wgt[None, :, :], axis=1)
    attn = (acc[:, :64] + acc[:, 64:]).T
    join = jnp.concatenate([up_ref[...], q, attn], axis=1)
    join_ref[...] = join
    ssum = jnp.sum(join, axis=0, keepdims=True)
    ssq = jnp.sum(join * join, axis=0, keepdims=True)
    part = jnp.concatenate([ssum, ssq], axis=0)

    @pl.when(pl.program_id(0) == 0)
    def _init():
        stats_ref[...] = jnp.zeros_like(stats_ref)

    stats_ref[...] += part


def _run_attn(hist, tgt, up, length3, W1T, Wqd, ba1c, W2T, ba2c,
              wa3s, ba3s):
    f32 = jnp.float32
    grid = (B // BB_A,)
    full = lambda shape: pl.BlockSpec(shape, lambda i: (0,) * len(shape))
    join, stats = pl.pallas_call(
        _attn_body,
        grid=grid,
        in_specs=[
            pl.BlockSpec((1, T, BB_A, 128), lambda i: (i, 0, 0, 0)),
            pl.BlockSpec((BB_A, 128), lambda i: (i, 0)),
            pl.BlockSpec((BB_A, 128), lambda i: (i, 0)),
            pl.BlockSpec((1, 1, 128), lambda i: (i, 0, 0)),
            full((80, 256)),
            full((128, 80)),
            full((80, 1)),
            full((40, 80)),
            full((40, 1)),
            full((40, 1)),
            full((1, 1)),
        ],
        out_specs=[
            pl.BlockSpec((BB_A, 384), lambda i: (i, 0)),
            pl.BlockSpec((2, 384), lambda i: (0, 0)),
        ],
        out_shape=[
            jax.ShapeDtypeStruct((B, 384), f32),
            jax.ShapeDtypeStruct((2, 384), f32),
        ],
        compiler_params=pltpu.CompilerParams(
            dimension_semantics=("arbitrary",)),
    )(hist.reshape(B // BB_A, T, BB_A, 128), up, tgt, length3, W1T, Wqd,
      ba1c, W2T, ba2c, wa3s, ba3s)
    return join, stats


def _tower_body(join_ref, stats_ref, gamma_ref, beta_ref, wf1_ref, bf1_ref,
                wf2_ref, bf2_ref, wf3_ref, bf3_ref, prob_ref, logit_ref):
    join = join_ref[...]
    stats = stats_ref[...]
    mean = stats[0:1, :] * (1.0 / B)
    var = stats[1:2, :] * (1.0 / B) - mean * mean
    scale = lax.rsqrt(var + 1e-3) * gamma_ref[...]
    xn = (join - mean) * scale + beta_ref[...]
    h = jnp.dot(xn, wf1_ref[...], preferred_element_type=jnp.float32)
    h = jnp.maximum(h + bf1_ref[...], 0.0)
    h = jnp.dot(h, wf2_ref[...], preferred_element_type=jnp.float32)
    h = jnp.maximum(h + bf2_ref[...], 0.0)
    logit = jnp.dot(h, wf3_ref[...], preferred_element_type=jnp.float32)
    logit = logit + bf3_ref[...]
    m = jnp.max(logit, axis=1, keepdims=True)
    e = jnp.exp(logit - m)
    prob_ref[...] = e / jnp.sum(e, axis=1, keepdims=True)
    logit_ref[...] = logit


def _run_tower(join, stats, gamma, beta, Wf1, bf1, Wf2, bf2, Wf3, bf3):
    f32 = jnp.float32
    grid = (B // BB_B,)
    full = lambda shape: pl.BlockSpec(shape, lambda i: (0,) * len(shape))
    prob, logit = pl.pallas_call(
        _tower_body,
        grid=grid,
        in_specs=[
            pl.BlockSpec((BB_B, 384), lambda i: (i, 0)),
            full((2, 384)),
            full((1, 384)),
            full((1, 384)),
            full((384, 200)),
            full((1, 200)),
            full((200, 80)),
            full((1, 80)),
            full((80, 2)),
            full((1, 2)),
        ],
        out_specs=[
            pl.BlockSpec((BB_B, 2), lambda i: (i, 0)),
            pl.BlockSpec((BB_B, 2), lambda i: (i, 0)),
        ],
        out_shape=[
            jax.ShapeDtypeStruct((B, 2), f32),
            jax.ShapeDtypeStruct((B, 2), f32),
        ],
    )(join, stats, gamma.reshape(1, 384), beta.reshape(1, 384),
      Wf1, bf1.reshape(1, 200), Wf2, bf2.reshape(1, 80),
      Wf3, bf3.reshape(1, 2))
    return prob, logit


def _tc_pipeline(up, tgt, hist,
                 Wa1, ba1, Wa2, ba2, Wa3, ba3, gamma, beta,
                 Wf1, bf1, Wf2, bf2, Wf3, bf3, length):
    Wq, Wh, Wd, Wp = Wa1[0:128], Wa1[128:256], Wa1[256:384], Wa1[384:512]
    W1T = jnp.concatenate([Wh - Wd, Wp], axis=0).T
    Wqd = Wq + Wd
    inv = 1.0 / jnp.sqrt(jnp.float32(128.0))
    ln = length.astype(jnp.int32).reshape(B // BB_A, BB_A)
    length3 = jnp.tile(ln, (1, 2)).reshape(B // BB_A, 1, 128)
    join, stats = _run_attn(
        hist, tgt, up, length3,
        W1T, Wqd, ba1.reshape(80, 1), Wa2.T, ba2.reshape(40, 1),
        Wa3 * inv, (ba3 * inv).reshape(1, 1))
    return _run_tower(join, stats, gamma, beta, Wf1, bf1, Wf2, bf2,
                      Wf3, bf3)


def kernel(E_user, E_gender, E_item, E_cate, Wa1, ba1, Wa2, ba2, Wa3, ba3,
           gamma, beta, Wf1, bf1, alpha1, Wf2, bf2, alpha2, Wf3, bf3,
           user_id, gender, target_item_id, target_cate_id,
           hist_item_id, hist_cate_id, length):
    i32 = jnp.int32
    EU = jnp.pad(E_user, ((0, 0), (0, 64)))
    EG = jnp.pad(E_gender, ((0, 0), (64, 0)))
    EI = jnp.pad(E_item, ((0, 0), (0, 64)))
    EC = jnp.pad(E_cate, ((0, 0), (64, 0)))

    def perm(idx):
        x = idx.astype(i32).reshape(B // BB_A, BB_A, T)
        return x.transpose(0, 2, 1).reshape(-1, 128)

    up, tgt, hist = _run_sc_gather(
        EU, EG, EI, EC,
        user_id.astype(i32).reshape(-1, 128),
        gender.astype(i32).reshape(-1, 128),
        target_item_id.astype(i32).reshape(-1, 128),
        target_cate_id.astype(i32).reshape(-1, 128),
        perm(hist_item_id),
        perm(hist_cate_id),
    )
    return _tc_pipeline(up, tgt, hist,
                        Wa1, ba1, Wa2, ba2, Wa3, ba3, gamma, beta,
                        Wf1, bf1, Wf2, bf2, Wf3, bf3, length)
